# baseline (speedup 1.0000x reference)
import functools

import jax
import jax.numpy as jnp
from jax.experimental import pallas as pl
from jax.experimental.pallas import tpu as pltpu

DEPTH = 7
PIX_PER_ROW = 128
ROW = PIX_PER_ROW * DEPTH
BLOCK_ROWS = 512


def _onehot_block(mask_ref, e_ref, out_ref):
    m = mask_ref[...].astype(jnp.bfloat16)
    m_exp = jax.lax.dot_general(
        m, e_ref[...],
        dimension_numbers=(((1,), (0,)), ((), ())),
        preferred_element_type=jnp.float32,
    )
    cls = jax.lax.broadcasted_iota(jnp.int32, m_exp.shape, 1) % DEPTH
    out_ref[...] = (m_exp == cls.astype(jnp.float32)).astype(jnp.float32)


@functools.partial(jax.jit, static_argnums=())
def _one_hot_flat(masks_flat2d, e_mat):
    n_rows = masks_flat2d.shape[0]
    grid = (n_rows // BLOCK_ROWS,)
    return pl.pallas_call(
        _onehot_block,
        grid=grid,
        in_specs=[
            pl.BlockSpec((BLOCK_ROWS, PIX_PER_ROW), lambda i: (i, 0)),
            pl.BlockSpec((PIX_PER_ROW, ROW), lambda i: (0, 0)),
        ],
        out_specs=pl.BlockSpec((BLOCK_ROWS, ROW), lambda i: (i, 0)),
        out_shape=jax.ShapeDtypeStruct((n_rows, ROW), jnp.float32),
    )(masks_flat2d, e_mat)


def kernel(images, masks, weights):
    b, h, w = masks.shape
    n_pix = b * h * w
    masks2d = masks.reshape(n_pix // PIX_PER_ROW, PIX_PER_ROW)
    e_mat = (jnp.arange(ROW, dtype=jnp.int32) // DEPTH
             == jnp.arange(PIX_PER_ROW, dtype=jnp.int32)[:, None]
             ).astype(jnp.bfloat16)
    oh_flat = _one_hot_flat(masks2d, e_mat)
    masks_oh = oh_flat.reshape(b, h, w, DEPTH)
    return (images, masks_oh, weights)


# speedup vs baseline: 27.2455x; 27.2455x over previous
import functools

import jax
import jax.numpy as jnp
from jax.experimental import pallas as pl

DEPTH = 7
H_BLK = 256


def _onehot_block(mask_ref, out_ref):
    m = mask_ref[0]
    for c in range(DEPTH):
        out_ref[0, c] = (m == c).astype(jnp.float32)


@jax.jit
def _one_hot_planes(masks):
    b, h, w = masks.shape
    grid = (b, h // H_BLK)
    return pl.pallas_call(
        _onehot_block,
        grid=grid,
        in_specs=[
            pl.BlockSpec((1, H_BLK, w), lambda i, j: (i, j, 0)),
        ],
        out_specs=pl.BlockSpec((1, DEPTH, H_BLK, w), lambda i, j: (i, 0, j, 0)),
        out_shape=jax.ShapeDtypeStruct((b, DEPTH, h, w), jnp.float32),
    )(masks)


def kernel(images, masks, weights):
    oh_planes = _one_hot_planes(masks)
    masks_oh = jnp.transpose(oh_planes, (0, 2, 3, 1))
    return (images, masks_oh, weights)
